# Initial kernel scaffold; baseline (speedup 1.0000x reference)
import jax
import jax.numpy as jnp
from jax.experimental import pallas as pl


def kernel(codes, tables):
    raise NotImplementedError("write your pallas kernel here")


# speedup vs baseline: 7.4598x; 7.4598x over previous
import functools

import jax
import jax.numpy as jnp
from jax import lax
from jax.experimental import pallas as pl
from jax.experimental.pallas import tpu as pltpu
from jax.experimental.pallas import tpu_sc as plsc

K = 8
V = 2048
D = 128
B = 16
T = 2048
NW = 32
P = B * T
PPW = P // NW
C = 64
NCH = PPW // C
LANES = 16
NV = D // LANES


def _make_kernel():
  mesh = plsc.VectorSubcoreMesh(core_axis_name="c", subcore_axis_name="s")

  @functools.partial(
      pl.kernel,
      mesh=mesh,
      out_type=jax.ShapeDtypeStruct((P, D), jnp.float32),
      scratch_types=[
          pltpu.VMEM((K, NCH, C), jnp.int32),
          pltpu.VMEM((K, C, D), jnp.float32),
          pltpu.VMEM((C, D), jnp.float32),
          pltpu.SemaphoreType.DMA,
      ],
  )
  def k(codes_hbm, ftab_hbm, out_hbm, idx_v, rows_v, acc_v, sem):
    wid = lax.axis_index("s") * 2 + lax.axis_index("c")
    pltpu.sync_copy(codes_hbm.at[wid], idx_v)

    def chunk_body(ci, _):
      cps = [
          pltpu.async_copy(ftab_hbm.at[idx_v.at[kk, ci]], rows_v.at[kk], sem)
          for kk in range(K)
      ]
      for cp in cps:
        cp.wait()

      def row_body(r, _):
        for c0 in range(NV):
          v = rows_v[0, r, pl.ds(c0 * LANES, LANES)]
          for kk in range(1, K):
            v = v + rows_v[kk, r, pl.ds(c0 * LANES, LANES)]
          acc_v[r, pl.ds(c0 * LANES, LANES)] = v
        return 0

      lax.fori_loop(0, C, row_body, 0)
      pltpu.sync_copy(acc_v, out_hbm.at[pl.ds(wid * PPW + ci * C, C)])
      return 0

    lax.fori_loop(0, NCH, chunk_body, 0)

  return k


_sc_kernel = _make_kernel()


def kernel(codes, tables):
  codes = codes.astype(jnp.int32)
  offs = (jnp.arange(K, dtype=jnp.int32) * V)[None, :, None]
  codes2 = (codes + offs).transpose(0, 2, 1).reshape(NW, PPW, K)
  codes2 = codes2.transpose(0, 2, 1).reshape(NW, K, NCH, C)
  ftab = tables.reshape(K * V, D)
  out = _sc_kernel(codes2, ftab)
  return out.reshape(B, T, D)


# speedup vs baseline: 10.6948x; 1.4337x over previous
import functools

import jax
import jax.numpy as jnp
from jax import lax
from jax.experimental import pallas as pl
from jax.experimental.pallas import tpu as pltpu
from jax.experimental.pallas import tpu_sc as plsc

K = 8
V = 2048
D = 128
B = 16
T = 2048
NW = 32
P = B * T
PPW = P // NW
C = 128
NCH = PPW // C


def _make_kernel():
  mesh = plsc.VectorSubcoreMesh(core_axis_name="c", subcore_axis_name="s")

  @functools.partial(
      pl.kernel,
      mesh=mesh,
      out_type=jax.ShapeDtypeStruct((P, D), jnp.float32),
      scratch_types=[
          pltpu.VMEM((K, NCH, C), jnp.int32),
          pltpu.VMEM((C, D), jnp.float32),
          pltpu.SemaphoreType.DMA,
      ],
  )
  def k(codes_hbm, ftab_hbm, out_hbm, idx_v, acc_v, sem):
    wid = lax.axis_index("s") * 2 + lax.axis_index("c")
    pltpu.sync_copy(codes_hbm.at[wid], idx_v)

    def chunk_body(ci, _):
      pltpu.async_copy(ftab_hbm.at[idx_v.at[0, ci]], acc_v, sem).wait()
      cps = [
          pltpu.async_copy(ftab_hbm.at[idx_v.at[kk, ci]], acc_v, sem, add=True)
          for kk in range(1, K)
      ]
      for cp in cps:
        cp.wait()
      pltpu.sync_copy(acc_v, out_hbm.at[pl.ds(wid * PPW + ci * C, C)])
      return 0

    lax.fori_loop(0, NCH, chunk_body, 0)

  return k


_sc_kernel = _make_kernel()


def kernel(codes, tables):
  codes = codes.astype(jnp.int32)
  offs = (jnp.arange(K, dtype=jnp.int32) * V)[None, :, None]
  codes2 = (codes + offs).transpose(0, 2, 1).reshape(NW, PPW, K)
  codes2 = codes2.transpose(0, 2, 1).reshape(NW, K, NCH, C)
  ftab = tables.reshape(K * V, D)
  out = _sc_kernel(codes2, ftab)
  return out.reshape(B, T, D)


# speedup vs baseline: 11.4481x; 1.0704x over previous
import functools

import jax
import jax.numpy as jnp
from jax import lax
from jax.experimental import pallas as pl
from jax.experimental.pallas import tpu as pltpu
from jax.experimental.pallas import tpu_sc as plsc

K = 8
V = 2048
D = 128
B = 16
T = 2048
NW = 32
P = B * T
PPW = P // NW
C = 128
NCH = PPW // C


def _make_kernel():
  mesh = plsc.VectorSubcoreMesh(core_axis_name="c", subcore_axis_name="s")

  @functools.partial(
      pl.kernel,
      mesh=mesh,
      out_type=jax.ShapeDtypeStruct((P, D), jnp.float32),
      scratch_types=[
          pltpu.VMEM((K, NCH, C), jnp.int32),
          pltpu.VMEM((C, D), jnp.float32),
          pltpu.VMEM((C, D), jnp.float32),
          pltpu.SemaphoreType.DMA,
          pltpu.SemaphoreType.DMA,
          pltpu.SemaphoreType.DMA,
          pltpu.SemaphoreType.DMA,
          pltpu.SemaphoreType.DMA,
          pltpu.SemaphoreType.DMA,
      ],
  )
  def k(codes_hbm, ftab_hbm, out_hbm, idx_v, acc0, acc1, sg0, sg1, sa0, sa1,
        so0, so1):
    wid = lax.axis_index("s") * 2 + lax.axis_index("c")
    acc = (acc0, acc1)
    sg = (sg0, sg1)
    sa = (sa0, sa1)
    so = (so0, so1)
    pltpu.sync_copy(codes_hbm.at[wid], idx_v)

    def plain(ci, p):
      pltpu.async_copy(ftab_hbm.at[idx_v.at[0, ci]], acc[p], sg[p])

    def drain_plain(ci, p):
      pltpu.make_async_copy(ftab_hbm.at[idx_v.at[0, ci]], acc[p], sg[p]).wait()

    plain(0, 0)
    for ci in range(NCH):
      p = ci % 2
      q = 1 - p
      drain_plain(ci, p)
      adds = [
          pltpu.async_copy(ftab_hbm.at[idx_v.at[kk, ci]], acc[p], sa[p],
                           add=True)
          for kk in range(1, K)
      ]
      if ci >= 2:
        pass
      if ci + 1 < NCH:
        if ci >= 1:
          pltpu.make_async_copy(acc[q], out_hbm.at[pl.ds(0, C)], so[q]).wait()
        plain(ci + 1, q)
      for cp in adds:
        cp.wait()
      pltpu.async_copy(acc[p], out_hbm.at[pl.ds(wid * PPW + ci * C, C)], so[p])
    pltpu.make_async_copy(
        acc[(NCH - 1) % 2],
        out_hbm.at[pl.ds(0, C)],
        so[(NCH - 1) % 2],
    ).wait()

  return k


_sc_kernel = _make_kernel()


def kernel(codes, tables):
  codes = codes.astype(jnp.int32)
  offs = (jnp.arange(K, dtype=jnp.int32) * V)[None, :, None]
  codes2 = (codes + offs).transpose(0, 2, 1).reshape(NW, PPW, K)
  codes2 = codes2.transpose(0, 2, 1).reshape(NW, K, NCH, C)
  ftab = tables.reshape(K * V, D)
  out = _sc_kernel(codes2, ftab)
  return out.reshape(B, T, D)
